# Initial kernel scaffold; baseline (speedup 1.0000x reference)
import jax
import jax.numpy as jnp
from jax.experimental import pallas as pl
from jax.experimental.pallas import tpu as pltpu

D_MODEL = 768
N_EXP = 8
D_FF = 1536
TOKENS = 2048
D_RIN = 64
TBLK = 256
NTB = TOKENS // TBLK


def _dense_body(hr, rr, wrh_r, wrc_r, br_r, w1_r, b1_r, w2_r, b2_r,
                out_r, bal_r, imp_acc, cnt_acc):
    t = pl.program_id(0)
    x = hr[...]
    logits = (jnp.dot(x, wrh_r[...], preferred_element_type=jnp.float32)
              + jnp.dot(rr[...], wrc_r[...], preferred_element_type=jnp.float32)
              + br_r[...])
    col = jax.lax.broadcasted_iota(jnp.int32, (TBLK, N_EXP), 1)
    m0 = jnp.max(logits, axis=1, keepdims=True)
    i0 = jnp.min(jnp.where(logits == m0, col, N_EXP), axis=1, keepdims=True)
    lm = jnp.where(col == i0, -jnp.inf, logits)
    m1 = jnp.max(lm, axis=1, keepdims=True)
    i1 = jnp.min(jnp.where(lm == m1, col, N_EXP), axis=1, keepdims=True)
    e1 = jnp.exp(m1 - m0)
    g0 = 1.0 / (1.0 + e1)
    g1 = e1 / (1.0 + e1)
    sel0 = jnp.where(col == i0, 1.0, 0.0)
    sel1 = jnp.where(col == i1, 1.0, 0.0)
    gate = g0 * sel0 + g1 * sel1

    p = jnp.exp(logits - m0)
    probs = p / jnp.sum(p, axis=1, keepdims=True)

    @pl.when(t == 0)
    def _():
        imp_acc[...] = jnp.zeros_like(imp_acc)
        cnt_acc[...] = jnp.zeros_like(cnt_acc)

    imp_acc[...] += jnp.sum(probs, axis=0, keepdims=True)
    cnt_acc[...] += jnp.sum(sel0 + sel1, axis=0, keepdims=True)

    xb = x.astype(jnp.bfloat16)
    acc = jnp.zeros((TBLK, D_MODEL), jnp.float32)
    for e in range(N_EXP):
        hdd = jnp.dot(xb, w1_r[e], preferred_element_type=jnp.float32) + b1_r[e]
        hdd = jax.nn.gelu(hdd)
        yo = jnp.dot(hdd.astype(jnp.bfloat16), w2_r[e],
                     preferred_element_type=jnp.float32) + b2_r[e]
        acc = acc + gate[:, e:e + 1] * yo
    out_r[...] = acc

    @pl.when(t == NTB - 1)
    def _():
        bal_r[0, 0] = (4.0 / (TOKENS * TOKENS)) * jnp.sum(imp_acc[...] * cnt_acc[...])


def kernel(h, router_in, Wr, br, W1, b1, W2, b2):
    wrh = Wr[:D_MODEL]
    wrc = jnp.zeros((D_RIN, N_EXP), Wr.dtype).at[D_RIN - 10:].set(Wr[D_MODEL:])
    w1b = W1.astype(jnp.bfloat16)
    w2b = W2.astype(jnp.bfloat16)
    out, bal = pl.pallas_call(
        _dense_body,
        grid=(NTB,),
        in_specs=[
            pl.BlockSpec((TBLK, D_MODEL), lambda t: (t, 0)),
            pl.BlockSpec((TBLK, D_RIN), lambda t: (t, 0)),
            pl.BlockSpec((D_MODEL, N_EXP), lambda t: (0, 0)),
            pl.BlockSpec((D_RIN, N_EXP), lambda t: (0, 0)),
            pl.BlockSpec((1, N_EXP), lambda t: (0, 0)),
            pl.BlockSpec((N_EXP, D_MODEL, D_FF), lambda t: (0, 0, 0)),
            pl.BlockSpec((N_EXP, D_FF), lambda t: (0, 0)),
            pl.BlockSpec((N_EXP, D_FF, D_MODEL), lambda t: (0, 0, 0)),
            pl.BlockSpec((N_EXP, D_MODEL), lambda t: (0, 0)),
        ],
        out_specs=[
            pl.BlockSpec((TBLK, D_MODEL), lambda t: (t, 0)),
            pl.BlockSpec((1, 1), lambda t: (0, 0)),
        ],
        out_shape=[
            jax.ShapeDtypeStruct((TOKENS, D_MODEL), jnp.float32),
            jax.ShapeDtypeStruct((1, 1), jnp.float32),
        ],
        scratch_shapes=[
            pltpu.VMEM((1, N_EXP), jnp.float32),
            pltpu.VMEM((1, N_EXP), jnp.float32),
        ],
    )(h, router_in, wrh, wrc, br.reshape(1, N_EXP), w1b, b1, w2b, b2)
    return out, bal[0, 0]


# speedup vs baseline: 1.1044x; 1.1044x over previous
import jax
import jax.numpy as jnp
from jax.experimental import pallas as pl
from jax.experimental.pallas import tpu as pltpu

D_MODEL = 768
N_EXP = 8
D_FF = 1536
TOKENS = 2048
D_RIN = 64
TBLK = 256
NTB = TOKENS // TBLK


def _dense_body(hr, rr, wrh_r, wrc_r, br_r, w1_r, b1_r, w2_r, b2_r,
                out_r, bal_r, imp_acc, cnt_acc):
    t = pl.program_id(0)
    x = hr[...]
    logits = (jnp.dot(x, wrh_r[...], preferred_element_type=jnp.float32)
              + jnp.dot(rr[...], wrc_r[...], preferred_element_type=jnp.float32)
              + br_r[...])
    col = jax.lax.broadcasted_iota(jnp.int32, (TBLK, N_EXP), 1)
    m0 = jnp.max(logits, axis=1, keepdims=True)
    i0 = jnp.min(jnp.where(logits == m0, col, N_EXP), axis=1, keepdims=True)
    lm = jnp.where(col == i0, -jnp.inf, logits)
    m1 = jnp.max(lm, axis=1, keepdims=True)
    i1 = jnp.min(jnp.where(lm == m1, col, N_EXP), axis=1, keepdims=True)
    e1 = jnp.exp(m1 - m0)
    g0 = 1.0 / (1.0 + e1)
    g1 = e1 / (1.0 + e1)
    sel0 = jnp.where(col == i0, 1.0, 0.0)
    sel1 = jnp.where(col == i1, 1.0, 0.0)
    gate = g0 * sel0 + g1 * sel1

    p = jnp.exp(logits - m0)
    probs = p / jnp.sum(p, axis=1, keepdims=True)

    @pl.when(t == 0)
    def _():
        imp_acc[...] = jnp.zeros_like(imp_acc)
        cnt_acc[...] = jnp.zeros_like(cnt_acc)

    imp_acc[...] += jnp.sum(probs, axis=0, keepdims=True)
    cnt_acc[...] += jnp.sum(sel0 + sel1, axis=0, keepdims=True)

    xb = x.astype(jnp.bfloat16)
    acc = jnp.zeros((TBLK, D_MODEL), jnp.float32)
    for e in range(N_EXP):
        hdd = jnp.dot(xb, w1_r[e], preferred_element_type=jnp.float32) + b1_r[e]
        hdd = jax.nn.gelu(hdd)
        yo = jnp.dot(hdd.astype(jnp.bfloat16), w2_r[e],
                     preferred_element_type=jnp.float32) + b2_r[e]
        acc = acc + gate[:, e:e + 1] * yo
    out_r[...] = acc

    @pl.when(t == NTB - 1)
    def _():
        bal_r[...] = ((4.0 / (TOKENS * TOKENS))
                      * jnp.sum(imp_acc[...] * cnt_acc[...])).reshape(1, 1)


def kernel(h, router_in, Wr, br, W1, b1, W2, b2):
    wrh = Wr[:D_MODEL]
    wrc = jnp.zeros((D_RIN, N_EXP), Wr.dtype).at[D_RIN - 10:].set(Wr[D_MODEL:])
    w1b = W1.astype(jnp.bfloat16)
    w2b = W2.astype(jnp.bfloat16)
    out, bal = pl.pallas_call(
        _dense_body,
        grid=(NTB,),
        in_specs=[
            pl.BlockSpec((TBLK, D_MODEL), lambda t: (t, 0)),
            pl.BlockSpec((TBLK, D_RIN), lambda t: (t, 0)),
            pl.BlockSpec((D_MODEL, N_EXP), lambda t: (0, 0)),
            pl.BlockSpec((D_RIN, N_EXP), lambda t: (0, 0)),
            pl.BlockSpec((1, N_EXP), lambda t: (0, 0)),
            pl.BlockSpec((N_EXP, D_MODEL, D_FF), lambda t: (0, 0, 0)),
            pl.BlockSpec((N_EXP, D_FF), lambda t: (0, 0)),
            pl.BlockSpec((N_EXP, D_FF, D_MODEL), lambda t: (0, 0, 0)),
            pl.BlockSpec((N_EXP, D_MODEL), lambda t: (0, 0)),
        ],
        out_specs=[
            pl.BlockSpec((TBLK, D_MODEL), lambda t: (t, 0)),
            pl.BlockSpec((1, 1), lambda t: (0, 0)),
        ],
        out_shape=[
            jax.ShapeDtypeStruct((TOKENS, D_MODEL), jnp.float32),
            jax.ShapeDtypeStruct((1, 1), jnp.float32),
        ],
        scratch_shapes=[
            pltpu.VMEM((1, N_EXP), jnp.float32),
            pltpu.VMEM((1, N_EXP), jnp.float32),
        ],
    )(h, router_in, wrh, wrc, br.reshape(1, N_EXP), w1b, b1, w2b, b2)
    return out, bal[0, 0]
